# baseline (speedup 1.0000x reference)
import functools

import jax
import jax.numpy as jnp
from jax import lax
from jax.experimental import pallas as pl
from jax.experimental.pallas import tpu as pltpu
from jax.experimental.pallas import tpu_sc as plsc

SCALE_ = 30.0
MARGIN_ = 0.2
B_ = 1024
C_ = 100000
LOG2E_ = 1.4426950408889634
K_ = SCALE_ * LOG2E_

NW_ = 32
CHR_ = 40
NCHK_ = 26
RPW_ = CHR_ * NCHK_
CSC_ = NW_ * RPW_
CT_ = C_ - CSC_
BCT_ = 2048
NLG_ = B_ // 16


def _fold8(v):
    acc = v[0:8, :]
    for k in range(1, v.shape[0] // 8):
        acc = acc + v[k * 8:(k + 1) * 8, :]
    return acc


def _tc_body(label_ref, cos_ref, s_ref, t_ref, sacc_ref, tacc_ref, *, nblk):
    j = pl.program_id(0)

    @pl.when(j == 0)
    def _init():
        sacc_ref[...] = jnp.zeros_like(sacc_ref)
        tacc_ref[...] = jnp.zeros_like(tacc_ref)

    x = cos_ref[...]
    cls = j * BCT_ + jax.lax.broadcasted_iota(jnp.int32, (BCT_, B_), 0)
    is_t = cls == label_ref[...]
    e = jnp.exp2(x * K_ - K_)

    @pl.when(j < nblk - 1)
    def _main():
        sacc_ref[...] += _fold8(e)
        tacc_ref[...] += _fold8(jnp.where(is_t, x, 0.0))

    @pl.when(j == nblk - 1)
    def _last():
        valid = cls < CT_
        sacc_ref[...] += _fold8(jnp.where(valid, e, 0.0))
        tacc_ref[...] += _fold8(jnp.where(is_t & valid, x, 0.0))
        s_ref[...] = jnp.sum(sacc_ref[...], axis=0, keepdims=True)
        t_ref[...] = jnp.sum(tacc_ref[...], axis=0, keepdims=True)


def _tc_main(cos_t, label):
    nblk = pl.cdiv(CT_, BCT_)
    return pl.pallas_call(
        functools.partial(_tc_body, nblk=nblk),
        grid=(nblk,),
        in_specs=[
            pl.BlockSpec((1, B_), lambda j: (0, 0)),
            pl.BlockSpec((BCT_, B_), lambda j: (j, 0)),
        ],
        out_specs=[
            pl.BlockSpec((1, B_), lambda j: (0, 0)),
            pl.BlockSpec((1, B_), lambda j: (0, 0)),
        ],
        out_shape=[
            jax.ShapeDtypeStruct((1, B_), jnp.float32),
            jax.ShapeDtypeStruct((1, B_), jnp.float32),
        ],
        scratch_shapes=[
            pltpu.VMEM((8, B_), jnp.float32),
            pltpu.VMEM((8, B_), jnp.float32),
        ],
    )(label.reshape(1, B_), cos_t)


def _sc_body(cos_hbm, lbl_hbm, s_out, t_out, buf0, buf1, lbl_v, sacc_v, tacc_v,
             sem0, sem1):
    wid = lax.axis_index("s") * 2 + lax.axis_index("c")
    row0 = CT_ + wid * RPW_
    pltpu.sync_copy(lbl_hbm, lbl_v)

    def _zero(g, _):
        sacc_v[pl.ds(g * 16, 16)] = jnp.zeros((16,), jnp.float32)
        tacc_v[pl.ds(g * 16, 16)] = jnp.zeros((16,), jnp.float32)
        return 0

    lax.fori_loop(0, NLG_, _zero, 0)

    bufs = (buf0, buf1)
    sems = (sem0, sem1)

    def start(c):
        return pltpu.async_copy(
            cos_hbm.at[pl.ds(row0 + c * CHR_, CHR_)], bufs[c % 2], sems[c % 2])

    pending = start(0)
    for c in range(NCHK_):
        if c + 1 < NCHK_:
            nxt = start(c + 1)
        pending.wait()
        pending = nxt if c + 1 < NCHK_ else None
        buf = bufs[c % 2]
        base_cls = row0 + c * CHR_

        def g_body(g, _):
            lbl16 = lbl_v[pl.ds(g * 16, 16)]
            lblrel = lbl16 - base_cls
            accs = [jnp.zeros((16,), jnp.float32) for _ in range(4)]
            taccs = [jnp.zeros((16,), jnp.float32) for _ in range(4)]
            for r in range(CHR_):
                x = buf[r, pl.ds(g * 16, 16)]
                accs[r % 4] = accs[r % 4] + jnp.exp(x * SCALE_ - SCALE_)
                taccs[r % 4] = taccs[r % 4] + jnp.where(lblrel == r, x, 0.0)
            plsc.addupdate(sacc_v.at[pl.ds(g * 16, 16)],
                           (accs[0] + accs[1]) + (accs[2] + accs[3]))
            plsc.addupdate(tacc_v.at[pl.ds(g * 16, 16)],
                           (taccs[0] + taccs[1]) + (taccs[2] + taccs[3]))
            return 0

        lax.fori_loop(0, NLG_, g_body, 0)

    pltpu.sync_copy(sacc_v, s_out.at[pl.ds(wid * B_, B_)])
    pltpu.sync_copy(tacc_v, t_out.at[pl.ds(wid * B_, B_)])


def _make_sc():
    mesh = plsc.VectorSubcoreMesh(core_axis_name="c", subcore_axis_name="s")
    return pl.kernel(
        _sc_body,
        out_type=(
            jax.ShapeDtypeStruct((NW_ * B_,), jnp.float32),
            jax.ShapeDtypeStruct((NW_ * B_,), jnp.float32),
        ),
        mesh=mesh,
        compiler_params=pltpu.CompilerParams(needs_layout_passes=False),
        scratch_types=[
            pltpu.VMEM((CHR_, B_), jnp.float32),
            pltpu.VMEM((CHR_, B_), jnp.float32),
            pltpu.VMEM((B_,), jnp.int32),
            pltpu.VMEM((B_,), jnp.float32),
            pltpu.VMEM((B_,), jnp.float32),
            pltpu.SemaphoreType.DMA,
            pltpu.SemaphoreType.DMA,
        ],
    )


def _combine_body(s_tc_ref, t_tc_ref, s_slab_ref, t_slab_ref, loss_ref):
    s = s_tc_ref[...] + jnp.sum(s_slab_ref[...], axis=0, keepdims=True)
    t = t_tc_ref[...] + jnp.sum(t_slab_ref[...], axis=0, keepdims=True)
    tm = (t - MARGIN_) * SCALE_
    s = s - jnp.exp2(t * K_ - K_) + jnp.exp2(tm * LOG2E_ - K_)
    nll = SCALE_ + jnp.log(s) - tm
    loss_ref[0, 0] = jnp.sum(nll) / B_


def _combine(s_tc, t_tc, s_slab, t_slab):
    return pl.pallas_call(
        _combine_body,
        grid=(1,),
        in_specs=[
            pl.BlockSpec((1, B_), lambda j: (0, 0)),
            pl.BlockSpec((1, B_), lambda j: (0, 0)),
            pl.BlockSpec((NW_, B_), lambda j: (0, 0)),
            pl.BlockSpec((NW_, B_), lambda j: (0, 0)),
        ],
        out_specs=pl.BlockSpec((1, 1), lambda j: (0, 0), memory_space=pltpu.SMEM),
        out_shape=jax.ShapeDtypeStruct((1, 1), jnp.float32),
    )(s_tc, t_tc, s_slab, t_slab)


def kernel(cosine, label):
    cos_t = cosine.T
    sc_fn = _make_sc()
    s_flat, t_flat = sc_fn(cos_t, label)
    s_tc, t_tc = _tc_main(cos_t, label)
    loss = _combine(s_tc, t_tc,
                    s_flat.reshape(NW_, B_), t_flat.reshape(NW_, B_))
    return loss[0, 0]


# speedup vs baseline: 1.0530x; 1.0530x over previous
import functools

import jax
import jax.numpy as jnp
from jax import lax
from jax.experimental import pallas as pl
from jax.experimental.pallas import tpu as pltpu
from jax.experimental.pallas import tpu_sc as plsc

SCALE_ = 30.0
MARGIN_ = 0.2
B_ = 1024
C_ = 100000
LOG2E_ = 1.4426950408889634
K_ = SCALE_ * LOG2E_

NW_ = 32
CHR_ = 40
NCHK_ = 24
RPW_ = CHR_ * NCHK_
CSC_ = NW_ * RPW_
CT_ = C_ - CSC_
BCT_ = 3072
NLG_ = B_ // 16


def _fold8(v):
    acc = v[0:8, :]
    for k in range(1, v.shape[0] // 8):
        acc = acc + v[k * 8:(k + 1) * 8, :]
    return acc


def _tc_body(label_ref, cos_ref, s_ref, t_ref, sacc_ref, tacc_ref, *, nblk):
    j = pl.program_id(0)

    @pl.when(j == 0)
    def _init():
        sacc_ref[...] = jnp.zeros_like(sacc_ref)
        tacc_ref[...] = jnp.zeros_like(tacc_ref)

    x = cos_ref[...]
    cls = j * BCT_ + jax.lax.broadcasted_iota(jnp.int32, (BCT_, B_), 0)
    is_t = cls == label_ref[...]
    e = jnp.exp2(x * K_ - K_)

    @pl.when(j < nblk - 1)
    def _main():
        sacc_ref[...] += _fold8(e)
        tacc_ref[...] += _fold8(jnp.where(is_t, x, 0.0))

    @pl.when(j == nblk - 1)
    def _last():
        valid = cls < CT_
        sacc_ref[...] += _fold8(jnp.where(valid, e, 0.0))
        tacc_ref[...] += _fold8(jnp.where(is_t & valid, x, 0.0))
        s_ref[...] = jnp.sum(sacc_ref[...], axis=0, keepdims=True)
        t_ref[...] = jnp.sum(tacc_ref[...], axis=0, keepdims=True)


def _tc_main(cos_t, label):
    nblk = pl.cdiv(CT_, BCT_)
    return pl.pallas_call(
        functools.partial(_tc_body, nblk=nblk),
        grid=(nblk,),
        in_specs=[
            pl.BlockSpec((1, B_), lambda j: (0, 0)),
            pl.BlockSpec((BCT_, B_), lambda j: (j, 0)),
        ],
        out_specs=[
            pl.BlockSpec((1, B_), lambda j: (0, 0)),
            pl.BlockSpec((1, B_), lambda j: (0, 0)),
        ],
        out_shape=[
            jax.ShapeDtypeStruct((1, B_), jnp.float32),
            jax.ShapeDtypeStruct((1, B_), jnp.float32),
        ],
        scratch_shapes=[
            pltpu.VMEM((8, B_), jnp.float32),
            pltpu.VMEM((8, B_), jnp.float32),
        ],
    )(label.reshape(1, B_), cos_t)


def _sc_body(cos_hbm, lbl_hbm, s_out, t_out, buf0, buf1, lbl_v, sacc_v, tacc_v,
             sem0, sem1):
    wid = lax.axis_index("s") * 2 + lax.axis_index("c")
    row0 = CT_ + wid * RPW_
    pltpu.sync_copy(lbl_hbm, lbl_v)

    def _zero(g, _):
        sacc_v[pl.ds(g * 16, 16)] = jnp.zeros((16,), jnp.float32)
        tacc_v[pl.ds(g * 16, 16)] = jnp.zeros((16,), jnp.float32)
        return 0

    lax.fori_loop(0, NLG_, _zero, 0)

    bufs = (buf0, buf1)
    sems = (sem0, sem1)

    def start(c):
        return pltpu.async_copy(
            cos_hbm.at[pl.ds(row0 + c * CHR_, CHR_)], bufs[c % 2], sems[c % 2])

    pending = start(0)
    for c in range(NCHK_):
        if c + 1 < NCHK_:
            nxt = start(c + 1)
        pending.wait()
        pending = nxt if c + 1 < NCHK_ else None
        buf = bufs[c % 2]
        base_cls = row0 + c * CHR_

        def g_body(g, _):
            lbl16 = lbl_v[pl.ds(g * 16, 16)]
            lblrel = lbl16 - base_cls
            accs = [jnp.zeros((16,), jnp.float32) for _ in range(4)]
            taccs = [jnp.zeros((16,), jnp.float32) for _ in range(4)]
            for r in range(CHR_):
                x = buf[r, pl.ds(g * 16, 16)]
                accs[r % 4] = accs[r % 4] + jnp.exp(x * SCALE_ - SCALE_)
                taccs[r % 4] = taccs[r % 4] + jnp.where(lblrel == r, x, 0.0)
            plsc.addupdate(sacc_v.at[pl.ds(g * 16, 16)],
                           (accs[0] + accs[1]) + (accs[2] + accs[3]))
            plsc.addupdate(tacc_v.at[pl.ds(g * 16, 16)],
                           (taccs[0] + taccs[1]) + (taccs[2] + taccs[3]))
            return 0

        lax.fori_loop(0, NLG_, g_body, 0)

    pltpu.sync_copy(sacc_v, s_out.at[pl.ds(wid * B_, B_)])
    pltpu.sync_copy(tacc_v, t_out.at[pl.ds(wid * B_, B_)])


def _make_sc():
    mesh = plsc.VectorSubcoreMesh(core_axis_name="c", subcore_axis_name="s")
    return pl.kernel(
        _sc_body,
        out_type=(
            jax.ShapeDtypeStruct((NW_ * B_,), jnp.float32),
            jax.ShapeDtypeStruct((NW_ * B_,), jnp.float32),
        ),
        mesh=mesh,
        compiler_params=pltpu.CompilerParams(needs_layout_passes=False),
        scratch_types=[
            pltpu.VMEM((CHR_, B_), jnp.float32),
            pltpu.VMEM((CHR_, B_), jnp.float32),
            pltpu.VMEM((B_,), jnp.int32),
            pltpu.VMEM((B_,), jnp.float32),
            pltpu.VMEM((B_,), jnp.float32),
            pltpu.SemaphoreType.DMA,
            pltpu.SemaphoreType.DMA,
        ],
    )


def _combine_body(s_tc_ref, t_tc_ref, s_slab_ref, t_slab_ref, loss_ref):
    s = s_tc_ref[...] + jnp.sum(s_slab_ref[...], axis=0, keepdims=True)
    t = t_tc_ref[...] + jnp.sum(t_slab_ref[...], axis=0, keepdims=True)
    tm = (t - MARGIN_) * SCALE_
    s = s - jnp.exp2(t * K_ - K_) + jnp.exp2(tm * LOG2E_ - K_)
    nll = SCALE_ + jnp.log(s) - tm
    loss_ref[0, 0] = jnp.sum(nll) / B_


def _combine(s_tc, t_tc, s_slab, t_slab):
    return pl.pallas_call(
        _combine_body,
        grid=(1,),
        in_specs=[
            pl.BlockSpec((1, B_), lambda j: (0, 0)),
            pl.BlockSpec((1, B_), lambda j: (0, 0)),
            pl.BlockSpec((NW_, B_), lambda j: (0, 0)),
            pl.BlockSpec((NW_, B_), lambda j: (0, 0)),
        ],
        out_specs=pl.BlockSpec((1, 1), lambda j: (0, 0), memory_space=pltpu.SMEM),
        out_shape=jax.ShapeDtypeStruct((1, 1), jnp.float32),
    )(s_tc, t_tc, s_slab, t_slab)


def kernel(cosine, label):
    cos_t = cosine.T
    sc_fn = _make_sc()
    s_flat, t_flat = sc_fn(cos_t, label)
    s_tc, t_tc = _tc_main(cos_t, label)
    loss = _combine(s_tc, t_tc,
                    s_flat.reshape(NW_, B_), t_flat.reshape(NW_, B_))
    return loss[0, 0]
